# Initial kernel scaffold; baseline (speedup 1.0000x reference)
import jax
import jax.numpy as jnp
from jax.experimental import pallas as pl


def kernel(atom_features, bond_features, pair_indices, kernel, bias):
    raise NotImplementedError("write your pallas kernel here")


# speedup vs baseline: 2.1524x; 2.1524x over previous
import functools

import jax
import jax.numpy as jnp
from jax import lax
from jax.experimental import pallas as pl
from jax.experimental.pallas import tpu as pltpu
from jax.experimental.pallas import tpu_sc as plsc

NC = 2
NS = 16
NW = NC * NS
C = 40
CL = 1000


def _tc_fold_body(x_ref, bond_ref, k2_ref, o_ref):
    x = x_ref[...]
    t = jnp.dot(x, k2_ref[...], preferred_element_type=jnp.float32)
    acc = t[:, 512:544]
    for b in range(16):
        acc = acc + bond_ref[:, b:b + 1] * t[:, b * 32:(b + 1) * 32]
    o_ref[...] = acc


def _tc_add_body(p_ref, o_ref):
    o_ref[...] = p_ref[0] + p_ref[1]


def _sc_gather(atom, src2, e, ad):
    nch = src2.shape[0] // NW
    epw = e // NW
    mesh = plsc.VectorSubcoreMesh(core_axis_name="c", subcore_axis_name="s")

    @functools.partial(
        pl.kernel,
        out_type=jax.ShapeDtypeStruct((e, ad), jnp.float32),
        mesh=mesh,
        scratch_types=[
            pltpu.VMEM((nch, C), jnp.int32),
            pltpu.VMEM((C, ad), jnp.float32),
            pltpu.SemaphoreType.DMA,
        ],
        compiler_params=pltpu.CompilerParams(use_tc_tiling_on_sc=False),
    )
    def k(atom_hbm, src_hbm, x_hbm, idx_v, rows_v, sem):
        cid = lax.axis_index("c")
        sid = lax.axis_index("s")
        wid = cid * NS + sid
        pltpu.sync_copy(src_hbm.at[pl.ds(wid * nch, nch)], idx_v)

        def body(j, carry):
            pltpu.async_copy(atom_hbm.at[idx_v.at[j]], rows_v, sem).wait()
            pltpu.sync_copy(rows_v, x_hbm.at[pl.ds(wid * epw + j * C, C)])
            return carry

        lax.fori_loop(0, nch, body, 0)

    return k(atom, src2)


def _sc_scatter(t, dst2, zeros, n, e, ad):
    nch = dst2.shape[0] // NW
    epw = e // NW
    rpc = n // NS
    mesh = plsc.VectorSubcoreMesh(core_axis_name="c", subcore_axis_name="s")

    @functools.partial(
        pl.kernel,
        out_type=jax.ShapeDtypeStruct((NC, n, ad), jnp.float32),
        mesh=mesh,
        scratch_types=[
            pltpu.VMEM((nch, C), jnp.int32),
            pltpu.VMEM((CL, ad), jnp.float32),
            pltpu.VMEM_SHARED((n, ad), jnp.float32),
            pltpu.SemaphoreType.DMA,
        ],
        compiler_params=pltpu.CompilerParams(use_tc_tiling_on_sc=False),
    )
    def k(t_hbm, dst_hbm, z_hbm, out_hbm, idx_v, rows_v, acc_sh, sem):
        cid = lax.axis_index("c")
        sid = lax.axis_index("s")
        wid = cid * NS + sid
        pltpu.sync_copy(z_hbm.at[pl.ds(sid * rpc, rpc)],
                        acc_sh.at[pl.ds(sid * rpc, rpc)])
        pltpu.sync_copy(dst_hbm.at[pl.ds(wid * nch, nch)], idx_v)
        plsc.subcore_barrier()

        def outer(g, carry):
            pltpu.async_copy(t_hbm.at[pl.ds(wid * epw + g * CL, CL)],
                             rows_v, sem).wait()

            def inner(j, c2):
                pltpu.sync_copy(rows_v.at[pl.ds(j * C, C)],
                                acc_sh.at[idx_v.at[g * (CL // C) + j]],
                                add=True)
                return c2

            lax.fori_loop(0, CL // C, inner, 0)
            return carry

        lax.fori_loop(0, epw // CL, outer, 0)
        plsc.subcore_barrier()
        pltpu.sync_copy(acc_sh.at[pl.ds(sid * rpc, rpc)],
                        out_hbm.at[cid, pl.ds(sid * rpc, rpc)])

    return k(t, dst2, zeros)


def kernel(atom_features, bond_features, pair_indices, kernel, bias):
    n, ad = atom_features.shape
    e, bd = bond_features.shape
    assert e % (NW * C) == 0 and e % (NW * CL) == 0 and n % NS == 0

    kr = kernel.reshape(bd, ad, ad)
    k2 = kr.transpose(2, 0, 1).reshape(ad, bd * ad)
    b2 = bias.reshape(ad, ad).T
    k2full = jnp.concatenate([k2, b2], axis=1)

    src2 = pair_indices[:, 1].astype(jnp.int32).reshape(e // C, C)
    dst2 = pair_indices[:, 0].astype(jnp.int32).reshape(e // C, C)

    x = _sc_gather(atom_features, src2, e, ad)

    be = 2000
    transformed = pl.pallas_call(
        _tc_fold_body,
        grid=(e // be,),
        in_specs=[
            pl.BlockSpec((be, ad), lambda i: (i, 0)),
            pl.BlockSpec((be, bd), lambda i: (i, 0)),
            pl.BlockSpec((ad, (bd + 1) * ad), lambda i: (0, 0)),
        ],
        out_specs=pl.BlockSpec((be, ad), lambda i: (i, 0)),
        out_shape=jax.ShapeDtypeStruct((e, ad), jnp.float32),
    )(x, bond_features, k2full)

    zeros = jnp.zeros((n, ad), jnp.float32)
    partials = _sc_scatter(transformed, dst2, zeros, n, e, ad)

    nb = 2000
    out = pl.pallas_call(
        _tc_add_body,
        grid=(n // nb,),
        in_specs=[pl.BlockSpec((NC, nb, ad), lambda i: (0, i, 0))],
        out_specs=pl.BlockSpec((nb, ad), lambda i: (i, 0)),
        out_shape=jax.ShapeDtypeStruct((n, ad), jnp.float32),
    )(partials)
    return out


# speedup vs baseline: 4.5587x; 2.1179x over previous
import functools

import jax
import jax.numpy as jnp
from jax import lax
from jax.experimental import pallas as pl
from jax.experimental.pallas import tpu as pltpu
from jax.experimental.pallas import tpu_sc as plsc

NC = 2
NS = 16
NW = NC * NS
C = 40
CL = 1000


def _tc_fold_body(x_ref, bond_ref, k2t_ref, o_ref):
    xt = x_ref[...].T.astype(jnp.bfloat16)
    tt = jnp.dot(k2t_ref[...].astype(jnp.bfloat16), xt,
                 preferred_element_type=jnp.float32)
    bt = bond_ref[...].T
    acc = tt[512:544, :]
    for b in range(16):
        acc = acc + bt[b:b + 1, :] * tt[b * 32:(b + 1) * 32, :]
    o_ref[...] = acc.T


def _tc_add_body(p_ref, o_ref):
    o_ref[...] = p_ref[0] + p_ref[1]


def _sc_gather(atom, src2, e, ad):
    nch = src2.shape[0] // NW
    epw = e // NW
    mesh = plsc.VectorSubcoreMesh(core_axis_name="c", subcore_axis_name="s")

    @functools.partial(
        pl.kernel,
        out_type=jax.ShapeDtypeStruct((e, ad), jnp.float32),
        mesh=mesh,
        scratch_types=[
            pltpu.VMEM((nch, C), jnp.int32),
            pltpu.VMEM((C, ad), jnp.float32),
            pltpu.SemaphoreType.DMA,
        ],
        compiler_params=pltpu.CompilerParams(use_tc_tiling_on_sc=False),
    )
    def k(atom_hbm, src_hbm, x_hbm, idx_v, rows_v, sem):
        cid = lax.axis_index("c")
        sid = lax.axis_index("s")
        wid = cid * NS + sid
        pltpu.sync_copy(src_hbm.at[pl.ds(wid * nch, nch)], idx_v)

        def body(j, carry):
            pltpu.async_copy(atom_hbm.at[idx_v.at[j]], rows_v, sem).wait()
            pltpu.sync_copy(rows_v, x_hbm.at[pl.ds(wid * epw + j * C, C)])
            return carry

        lax.fori_loop(0, nch, body, 0)

    return k(atom, src2)


def _sc_scatter(t, dst2, zeros, n, e, ad):
    nch = dst2.shape[0] // NW
    epw = e // NW
    rpc = n // NS
    mesh = plsc.VectorSubcoreMesh(core_axis_name="c", subcore_axis_name="s")

    @functools.partial(
        pl.kernel,
        out_type=jax.ShapeDtypeStruct((NC, n, ad), jnp.float32),
        mesh=mesh,
        scratch_types=[
            pltpu.VMEM((nch, C), jnp.int32),
            pltpu.VMEM((CL, ad), jnp.float32),
            pltpu.VMEM_SHARED((n, ad), jnp.float32),
            pltpu.SemaphoreType.DMA,
        ],
        compiler_params=pltpu.CompilerParams(use_tc_tiling_on_sc=False),
    )
    def k(t_hbm, dst_hbm, z_hbm, out_hbm, idx_v, rows_v, acc_sh, sem):
        cid = lax.axis_index("c")
        sid = lax.axis_index("s")
        wid = cid * NS + sid
        pltpu.sync_copy(z_hbm.at[pl.ds(sid * rpc, rpc)],
                        acc_sh.at[pl.ds(sid * rpc, rpc)])
        pltpu.sync_copy(dst_hbm.at[pl.ds(wid * nch, nch)], idx_v)
        plsc.subcore_barrier()

        def outer(g, carry):
            pltpu.async_copy(t_hbm.at[pl.ds(wid * epw + g * CL, CL)],
                             rows_v, sem).wait()

            def inner(j, c2):
                pltpu.sync_copy(rows_v.at[pl.ds(j * C, C)],
                                acc_sh.at[idx_v.at[g * (CL // C) + j]],
                                add=True)
                return c2

            lax.fori_loop(0, CL // C, inner, 0)
            return carry

        lax.fori_loop(0, epw // CL, outer, 0)
        plsc.subcore_barrier()
        pltpu.sync_copy(acc_sh.at[pl.ds(sid * rpc, rpc)],
                        out_hbm.at[cid, pl.ds(sid * rpc, rpc)])

    return k(t, dst2, zeros)


def kernel(atom_features, bond_features, pair_indices, kernel, bias):
    n, ad = atom_features.shape
    e, bd = bond_features.shape
    assert e % (NW * C) == 0 and e % (NW * CL) == 0 and n % NS == 0

    kr = kernel.reshape(bd, ad, ad)
    k2 = kr.transpose(2, 0, 1).reshape(ad, bd * ad)
    b2 = bias.reshape(ad, ad).T
    k2t = jnp.concatenate([k2, b2], axis=1).T

    src2 = pair_indices[:, 1].astype(jnp.int32).reshape(e // C, C)
    dst2 = pair_indices[:, 0].astype(jnp.int32).reshape(e // C, C)

    x = _sc_gather(atom_features, src2, e, ad)

    be = 2000
    transformed = pl.pallas_call(
        _tc_fold_body,
        grid=(e // be,),
        in_specs=[
            pl.BlockSpec((be, ad), lambda i: (i, 0)),
            pl.BlockSpec((be, bd), lambda i: (i, 0)),
            pl.BlockSpec(((bd + 1) * ad, ad), lambda i: (0, 0)),
        ],
        out_specs=pl.BlockSpec((be, ad), lambda i: (i, 0)),
        out_shape=jax.ShapeDtypeStruct((e, ad), jnp.float32),
    )(x, bond_features, k2t)

    zeros = jnp.zeros((n, ad), jnp.float32)
    partials = _sc_scatter(transformed, dst2, zeros, n, e, ad)

    nb = 2000
    out = pl.pallas_call(
        _tc_add_body,
        grid=(n // nb,),
        in_specs=[pl.BlockSpec((NC, nb, ad), lambda i: (0, i, 0))],
        out_specs=pl.BlockSpec((nb, ad), lambda i: (i, 0)),
        out_shape=jax.ShapeDtypeStruct((n, ad), jnp.float32),
    )(partials)
    return out


# speedup vs baseline: 5.7608x; 1.2637x over previous
import functools

import jax
import jax.numpy as jnp
from jax import lax
from jax.experimental import pallas as pl
from jax.experimental.pallas import tpu as pltpu
from jax.experimental.pallas import tpu_sc as plsc

NC = 2
NS = 16
NW = NC * NS
CQ = 1000


def _tc_fold_body(x_ref, bond_ref, k2t_ref, o_ref):
    xt = x_ref[...].T.astype(jnp.bfloat16)
    tt = jnp.dot(k2t_ref[...].astype(jnp.bfloat16), xt,
                 preferred_element_type=jnp.float32)
    bt = bond_ref[...].T
    acc = tt[512:544, :]
    for b in range(16):
        acc = acc + bt[b:b + 1, :] * tt[b * 32:(b + 1) * 32, :]
    o_ref[...] = acc.T


def _tc_add_body(p_ref, o_ref):
    o_ref[...] = p_ref[0] + p_ref[1]


def _sc_gather(atom, src, e, ad):
    epw = e // NW
    nq = epw // CQ
    mesh = plsc.VectorSubcoreMesh(core_axis_name="c", subcore_axis_name="s")

    @functools.partial(
        pl.kernel,
        out_type=jax.ShapeDtypeStruct((e, ad), jnp.float32),
        mesh=mesh,
        scratch_types=[pltpu.VMEM((CQ,), jnp.int32)] * nq
        + [pltpu.VMEM((2, CQ, ad), jnp.float32)]
        + [pltpu.SemaphoreType.DMA] * 2,
        compiler_params=pltpu.CompilerParams(use_tc_tiling_on_sc=False),
    )
    def k(atom_hbm, src_hbm, x_hbm, *rest):
        idx = rest[:nq]
        rows_v = rest[nq]
        sems = rest[nq + 1:]
        cid = lax.axis_index("c")
        sid = lax.axis_index("s")
        wid = cid * NS + sid
        base = wid * epw
        for q in range(nq):
            pltpu.sync_copy(src_hbm.at[pl.ds(base + q * CQ, CQ)], idx[q])
        pltpu.async_copy(atom_hbm.at[idx[0]], rows_v.at[0], sems[0])
        for q in range(nq):
            if q + 1 < nq:
                pltpu.async_copy(atom_hbm.at[idx[q + 1]],
                                 rows_v.at[(q + 1) % 2], sems[(q + 1) % 2])
            pltpu.make_async_copy(atom_hbm.at[idx[q]],
                                  rows_v.at[q % 2], sems[q % 2]).wait()
            pltpu.sync_copy(rows_v.at[q % 2],
                            x_hbm.at[pl.ds(base + q * CQ, CQ)])

    return k(atom, src)


def _sc_scatter(t, dst, zeros, n, e, ad):
    epw = e // NW
    rpc = n // NS
    mesh = plsc.VectorSubcoreMesh(core_axis_name="c", subcore_axis_name="s")

    nq = epw // CQ

    @functools.partial(
        pl.kernel,
        out_type=jax.ShapeDtypeStruct((NC, n, ad), jnp.float32),
        mesh=mesh,
        scratch_types=[pltpu.VMEM((CQ,), jnp.int32)] * nq
        + [pltpu.VMEM((2, CQ, ad), jnp.float32)]
        + [pltpu.VMEM_SHARED((n, ad), jnp.float32)]
        + [pltpu.SemaphoreType.DMA] * 2,
        compiler_params=pltpu.CompilerParams(use_tc_tiling_on_sc=False),
    )
    def k(t_hbm, dst_hbm, z_hbm, out_hbm, *rest):
        idx = rest[:nq]
        rows_v = rest[nq]
        acc_sh = rest[nq + 1]
        sems = rest[nq + 2:]
        cid = lax.axis_index("c")
        sid = lax.axis_index("s")
        wid = cid * NS + sid
        base = wid * epw
        pltpu.sync_copy(z_hbm.at[pl.ds(sid * rpc, rpc)],
                        acc_sh.at[pl.ds(sid * rpc, rpc)])
        for q in range(nq):
            pltpu.sync_copy(dst_hbm.at[pl.ds(base + q * CQ, CQ)], idx[q])
        plsc.subcore_barrier()
        pltpu.async_copy(t_hbm.at[pl.ds(base, CQ)], rows_v.at[0], sems[0])
        for q in range(nq):
            if q + 1 < nq:
                pltpu.async_copy(t_hbm.at[pl.ds(base + (q + 1) * CQ, CQ)],
                                 rows_v.at[(q + 1) % 2], sems[(q + 1) % 2])
            pltpu.make_async_copy(t_hbm.at[pl.ds(base + q * CQ, CQ)],
                                  rows_v.at[q % 2], sems[q % 2]).wait()
            pltpu.sync_copy(rows_v.at[q % 2], acc_sh.at[idx[q]], add=True)
        plsc.subcore_barrier()
        pltpu.sync_copy(acc_sh.at[pl.ds(sid * rpc, rpc)],
                        out_hbm.at[cid, pl.ds(sid * rpc, rpc)])

    return k(t, dst, zeros)


def kernel(atom_features, bond_features, pair_indices, kernel, bias):
    n, ad = atom_features.shape
    e, bd = bond_features.shape
    assert e % (NW * CQ) == 0 and CQ % 8 == 0 and n % NS == 0

    kr = kernel.reshape(bd, ad, ad)
    k2 = kr.transpose(2, 0, 1).reshape(ad, bd * ad)
    b2 = bias.reshape(ad, ad).T
    k2t = jnp.concatenate([k2, b2], axis=1).T

    pairs = pair_indices.astype(jnp.int32)
    dst = pairs[:, 0]
    src = pairs[:, 1]

    x = _sc_gather(atom_features, src, e, ad)

    be = 4000
    transformed = pl.pallas_call(
        _tc_fold_body,
        grid=(e // be,),
        in_specs=[
            pl.BlockSpec((be, ad), lambda i: (i, 0)),
            pl.BlockSpec((be, bd), lambda i: (i, 0)),
            pl.BlockSpec(((bd + 1) * ad, ad), lambda i: (0, 0)),
        ],
        out_specs=pl.BlockSpec((be, ad), lambda i: (i, 0)),
        out_shape=jax.ShapeDtypeStruct((e, ad), jnp.float32),
    )(x, bond_features, k2t)

    zeros = jnp.zeros((n, ad), jnp.float32)
    partials = _sc_scatter(transformed, dst, zeros, n, e, ad)

    nb = 2000
    out = pl.pallas_call(
        _tc_add_body,
        grid=(n // nb,),
        in_specs=[pl.BlockSpec((NC, nb, ad), lambda i: (0, i, 0))],
        out_specs=pl.BlockSpec((nb, ad), lambda i: (i, 0)),
        out_shape=jax.ShapeDtypeStruct((n, ad), jnp.float32),
    )(partials)
    return out


# speedup vs baseline: 6.1522x; 1.0679x over previous
import functools

import jax
import jax.numpy as jnp
from jax import lax
from jax.experimental import pallas as pl
from jax.experimental.pallas import tpu as pltpu
from jax.experimental.pallas import tpu_sc as plsc

NC = 2
NS = 16
NW = NC * NS
CQ = 1000


def _tc_fold_body(x_ref, bond_ref, k2t_ref, o_ref):
    xt = x_ref[...].T.astype(jnp.bfloat16)
    tt = jnp.dot(k2t_ref[...].astype(jnp.bfloat16), xt,
                 preferred_element_type=jnp.float32)
    bt = bond_ref[...].T
    acc = tt[512:544, :]
    for b in range(16):
        acc = acc + bt[b:b + 1, :] * tt[b * 32:(b + 1) * 32, :]
    o_ref[...] = acc.T


def _tc_add_body(p_ref, o_ref):
    o_ref[...] = p_ref[0] + p_ref[1]


def _sc_gather(atom, pt, e, ad):
    epw = e // NW
    nq = epw // CQ
    mesh = plsc.VectorSubcoreMesh(core_axis_name="c", subcore_axis_name="s")

    @functools.partial(
        pl.kernel,
        out_type=jax.ShapeDtypeStruct((e, ad), jnp.float32),
        mesh=mesh,
        scratch_types=[pltpu.VMEM((CQ,), jnp.int32)] * nq
        + [pltpu.VMEM((2, CQ, ad), jnp.float32)]
        + [pltpu.SemaphoreType.DMA] * 2,
        compiler_params=pltpu.CompilerParams(use_tc_tiling_on_sc=False),
    )
    def k(atom_hbm, pt_hbm, x_hbm, *rest):
        idx = rest[:nq]
        rows_v = rest[nq]
        sems = rest[nq + 1:]
        cid = lax.axis_index("c")
        sid = lax.axis_index("s")
        wid = cid * NS + sid
        base = wid * epw
        for q in range(nq):
            pltpu.sync_copy(pt_hbm.at[1, pl.ds(base + q * CQ, CQ)], idx[q])
        pltpu.async_copy(atom_hbm.at[idx[0]], rows_v.at[0], sems[0])
        for q in range(nq):
            if q + 1 < nq:
                pltpu.async_copy(atom_hbm.at[idx[q + 1]],
                                 rows_v.at[(q + 1) % 2], sems[(q + 1) % 2])
            pltpu.make_async_copy(atom_hbm.at[idx[q]],
                                  rows_v.at[q % 2], sems[q % 2]).wait()
            pltpu.sync_copy(rows_v.at[q % 2],
                            x_hbm.at[pl.ds(base + q * CQ, CQ)])

    return k(atom, pt)


def _sc_scatter(t, pt, zeros, n, e, ad):
    epw = e // NW
    rpc = n // NS
    mesh = plsc.VectorSubcoreMesh(core_axis_name="c", subcore_axis_name="s")

    nq = epw // CQ

    @functools.partial(
        pl.kernel,
        out_type=jax.ShapeDtypeStruct((NC, n, ad), jnp.float32),
        mesh=mesh,
        scratch_types=[pltpu.VMEM((CQ,), jnp.int32)] * nq
        + [pltpu.VMEM((2, CQ, ad), jnp.float32)]
        + [pltpu.VMEM_SHARED((n, ad), jnp.float32)]
        + [pltpu.SemaphoreType.DMA] * 2,
        compiler_params=pltpu.CompilerParams(use_tc_tiling_on_sc=False),
    )
    def k(t_hbm, pt_hbm, z_hbm, out_hbm, *rest):
        idx = rest[:nq]
        rows_v = rest[nq]
        acc_sh = rest[nq + 1]
        sems = rest[nq + 2:]
        cid = lax.axis_index("c")
        sid = lax.axis_index("s")
        wid = cid * NS + sid
        base = wid * epw
        pltpu.sync_copy(z_hbm.at[pl.ds(sid * rpc, rpc)],
                        acc_sh.at[pl.ds(sid * rpc, rpc)])
        for q in range(nq):
            pltpu.sync_copy(pt_hbm.at[0, pl.ds(base + q * CQ, CQ)], idx[q])
        plsc.subcore_barrier()
        pltpu.async_copy(t_hbm.at[pl.ds(base, CQ)], rows_v.at[0], sems[0])
        for q in range(nq):
            if q + 1 < nq:
                pltpu.async_copy(t_hbm.at[pl.ds(base + (q + 1) * CQ, CQ)],
                                 rows_v.at[(q + 1) % 2], sems[(q + 1) % 2])
            pltpu.make_async_copy(t_hbm.at[pl.ds(base + q * CQ, CQ)],
                                  rows_v.at[q % 2], sems[q % 2]).wait()
            pltpu.sync_copy(rows_v.at[q % 2], acc_sh.at[idx[q]], add=True)
        plsc.subcore_barrier()
        pltpu.sync_copy(acc_sh.at[pl.ds(sid * rpc, rpc)],
                        out_hbm.at[cid, pl.ds(sid * rpc, rpc)])

    return k(t, pt, zeros)


def kernel(atom_features, bond_features, pair_indices, kernel, bias):
    n, ad = atom_features.shape
    e, bd = bond_features.shape
    assert e % (NW * CQ) == 0 and CQ % 8 == 0 and n % NS == 0

    kr = kernel.reshape(bd, ad, ad)
    k2 = kr.transpose(2, 0, 1).reshape(ad, bd * ad)
    b2 = bias.reshape(ad, ad).T
    k2t = jnp.concatenate([k2, b2], axis=1).T

    pt = pair_indices.astype(jnp.int32).T

    x = _sc_gather(atom_features, pt, e, ad)

    be = 8000
    transformed = pl.pallas_call(
        _tc_fold_body,
        grid=(e // be,),
        in_specs=[
            pl.BlockSpec((be, ad), lambda i: (i, 0)),
            pl.BlockSpec((be, bd), lambda i: (i, 0)),
            pl.BlockSpec(((bd + 1) * ad, ad), lambda i: (0, 0)),
        ],
        out_specs=pl.BlockSpec((be, ad), lambda i: (i, 0)),
        out_shape=jax.ShapeDtypeStruct((e, ad), jnp.float32),
    )(x, bond_features, k2t)

    zeros = jnp.zeros((n, ad), jnp.float32)
    partials = _sc_scatter(transformed, pt, zeros, n, e, ad)

    nb = 2000
    out = pl.pallas_call(
        _tc_add_body,
        grid=(n // nb,),
        in_specs=[pl.BlockSpec((NC, nb, ad), lambda i: (0, i, 0))],
        out_specs=pl.BlockSpec((nb, ad), lambda i: (i, 0)),
        out_shape=jax.ShapeDtypeStruct((n, ad), jnp.float32),
    )(partials)
    return out


# speedup vs baseline: 8.0450x; 1.3077x over previous
import functools

import jax
import jax.numpy as jnp
from jax import lax
from jax.experimental import pallas as pl
from jax.experimental.pallas import tpu as pltpu
from jax.experimental.pallas import tpu_sc as plsc

NC = 2
NS = 16
NW = NC * NS
CQ = 1000


def _tc_fold_body(x4_ref, bt4_ref, k4_ref, o_ref):
    x4t = x4_ref[...].T.astype(jnp.bfloat16)
    tt4 = jnp.dot(k4_ref[...].astype(jnp.bfloat16), x4t,
                  preferred_element_type=jnp.float32)
    bt4 = bt4_ref[0]
    accs = []
    for g in range(4):
        a = tt4[g * 544 + 512:g * 544 + 544, :]
        for b in range(16):
            a = a + (bt4[g * 16 + b:g * 16 + b + 1, :]
                     * tt4[g * 544 + b * 32:g * 544 + (b + 1) * 32, :])
        accs.append(a)
    o_ref[...] = jnp.concatenate(accs, axis=0).T


def _tc_add_body(p_ref, o_ref):
    o_ref[...] = p_ref[0] + p_ref[1]


def _sc_gather(atom, pt, e, ad):
    epw = e // NW
    nq = epw // CQ
    mesh = plsc.VectorSubcoreMesh(core_axis_name="c", subcore_axis_name="s")

    @functools.partial(
        pl.kernel,
        out_type=jax.ShapeDtypeStruct((e, ad), jnp.float32),
        mesh=mesh,
        scratch_types=[pltpu.VMEM((CQ,), jnp.int32)] * nq
        + [pltpu.VMEM((2, CQ, ad), jnp.float32)]
        + [pltpu.SemaphoreType.DMA] * 2,
        compiler_params=pltpu.CompilerParams(use_tc_tiling_on_sc=False),
    )
    def k(atom_hbm, pt_hbm, x_hbm, *rest):
        idx = rest[:nq]
        rows_v = rest[nq]
        sems = rest[nq + 1:]
        cid = lax.axis_index("c")
        sid = lax.axis_index("s")
        wid = cid * NS + sid
        base = wid * epw
        for q in range(nq):
            pltpu.sync_copy(pt_hbm.at[1, pl.ds(base + q * CQ, CQ)], idx[q])
        pltpu.async_copy(atom_hbm.at[idx[0]], rows_v.at[0], sems[0])
        for q in range(nq):
            if q + 1 < nq:
                pltpu.async_copy(atom_hbm.at[idx[q + 1]],
                                 rows_v.at[(q + 1) % 2], sems[(q + 1) % 2])
            pltpu.make_async_copy(atom_hbm.at[idx[q]],
                                  rows_v.at[q % 2], sems[q % 2]).wait()
            pltpu.sync_copy(rows_v.at[q % 2],
                            x_hbm.at[pl.ds(base + q * CQ, CQ)])

    return k(atom, pt)


def _sc_scatter(t, pt, zeros, n, e, ad):
    epw = e // NW
    rpc = n // NS
    mesh = plsc.VectorSubcoreMesh(core_axis_name="c", subcore_axis_name="s")

    nq = epw // CQ

    @functools.partial(
        pl.kernel,
        out_type=jax.ShapeDtypeStruct((NC, n, ad), jnp.float32),
        mesh=mesh,
        scratch_types=[pltpu.VMEM((CQ,), jnp.int32)] * nq
        + [pltpu.VMEM((2, CQ, ad), jnp.float32)]
        + [pltpu.VMEM_SHARED((n, ad), jnp.float32)]
        + [pltpu.SemaphoreType.DMA] * 2,
        compiler_params=pltpu.CompilerParams(use_tc_tiling_on_sc=False),
    )
    def k(t_hbm, pt_hbm, z_hbm, out_hbm, *rest):
        idx = rest[:nq]
        rows_v = rest[nq]
        acc_sh = rest[nq + 1]
        sems = rest[nq + 2:]
        cid = lax.axis_index("c")
        sid = lax.axis_index("s")
        wid = cid * NS + sid
        base = wid * epw
        pltpu.sync_copy(z_hbm.at[pl.ds(sid * rpc, rpc)],
                        acc_sh.at[pl.ds(sid * rpc, rpc)])
        for q in range(nq):
            pltpu.sync_copy(pt_hbm.at[0, pl.ds(base + q * CQ, CQ)], idx[q])
        plsc.subcore_barrier()
        pltpu.async_copy(t_hbm.at[pl.ds(base, CQ)], rows_v.at[0], sems[0])
        for q in range(nq):
            if q + 1 < nq:
                pltpu.async_copy(t_hbm.at[pl.ds(base + (q + 1) * CQ, CQ)],
                                 rows_v.at[(q + 1) % 2], sems[(q + 1) % 2])
            pltpu.make_async_copy(t_hbm.at[pl.ds(base + q * CQ, CQ)],
                                  rows_v.at[q % 2], sems[q % 2]).wait()
            pltpu.sync_copy(rows_v.at[q % 2], acc_sh.at[idx[q]], add=True)
        plsc.subcore_barrier()
        pltpu.sync_copy(acc_sh.at[pl.ds(sid * rpc, rpc)],
                        out_hbm.at[cid, pl.ds(sid * rpc, rpc)])

    return k(t, pt, zeros)


def kernel(atom_features, bond_features, pair_indices, kernel, bias):
    n, ad = atom_features.shape
    e, bd = bond_features.shape
    assert e % (NW * CQ) == 0 and CQ % 8 == 0 and n % NS == 0

    kr = kernel.reshape(bd, ad, ad)
    k2 = kr.transpose(2, 0, 1).reshape(ad, bd * ad)
    b2 = bias.reshape(ad, ad).T
    k2t = jnp.concatenate([k2, b2], axis=1).T
    k4 = jnp.kron(jnp.eye(4, dtype=jnp.float32), k2t)

    pt = pair_indices.astype(jnp.int32).T

    x = _sc_gather(atom_features, pt, e, ad)

    be = 4000
    x4 = x.reshape(e // 4, 4 * ad)
    bt43 = bond_features.reshape(e // be, be // 4, 4 * bd).transpose(0, 2, 1)
    t4 = pl.pallas_call(
        _tc_fold_body,
        grid=(e // be,),
        in_specs=[
            pl.BlockSpec((be // 4, 4 * ad), lambda i: (i, 0)),
            pl.BlockSpec((1, 4 * bd, be // 4), lambda i: (i, 0, 0)),
            pl.BlockSpec((4 * (bd + 1) * ad, 4 * ad), lambda i: (0, 0)),
        ],
        out_specs=pl.BlockSpec((be // 4, 4 * ad), lambda i: (i, 0)),
        out_shape=jax.ShapeDtypeStruct((e // 4, 4 * ad), jnp.float32),
    )(x4, bt43, k4)
    transformed = t4.reshape(e, ad)

    zeros = jnp.zeros((n, ad), jnp.float32)
    partials = _sc_scatter(transformed, pt, zeros, n, e, ad)

    p4 = partials.reshape(NC, n * ad // 128, 128)
    nb = n * ad // 128
    out4 = pl.pallas_call(
        _tc_add_body,
        grid=(1,),
        in_specs=[pl.BlockSpec((NC, nb, 128), lambda i: (0, 0, 0))],
        out_specs=pl.BlockSpec((nb, 128), lambda i: (0, 0)),
        out_shape=jax.ShapeDtypeStruct((nb, 128), jnp.float32),
    )(p4)
    return out4.reshape(n, ad)
